# Initial kernel scaffold; baseline (speedup 1.0000x reference)
import jax
import jax.numpy as jnp
from jax.experimental import pallas as pl


def kernel(xyz, points, W_kernel, bn1_gamma, bn1_beta, bn1_mean, bn1_var, W_agg, bn2_gamma, bn2_beta, bn2_mean, bn2_var, W_lin, b_lin):
    raise NotImplementedError("write your pallas kernel here")


# speedup vs baseline: 5.8723x; 5.8723x over previous
import jax
import jax.numpy as jnp
from jax.experimental import pallas as pl
from jax.experimental.pallas import tpu as pltpu
from jax.experimental.pallas import tpu_sc as plsc

EPS = 1e-5
LEAKY = 0.1
B, N, DF, K, CIN, O = 4, 4096, 13, 32, 16, 32
S, L = 32, 128
M = 128
M2 = 128
R = 4
INF = 3.0e38
HIGH = jax.lax.Precision.HIGHEST


def _leaky(x):
    return jnp.where(x >= 0, x, LEAKY * x)


def _knn_kernel(q_ref, xt_ref, idx_ref):
    q = q_ref[0]
    xt = xt_ref[0]
    qb = q.astype(jnp.bfloat16).astype(jnp.float32)
    xb = xt.astype(jnp.bfloat16).astype(jnp.float32)
    dot = qb[0][:, None] * xb[0][None, :]
    dot = dot + qb[1][:, None] * xb[1][None, :]
    dot = dot + qb[2][:, None] * xb[2][None, :]
    sqm = jnp.sum(q * q, axis=0)[:, None]
    sqj = jnp.sum(xt * xt, axis=0)[None, :]
    d = (-2.0 * dot + sqm) + sqj
    d3 = d.reshape(M, S, L)

    lv = [jnp.full((M, L), INF, jnp.float32) for _ in range(R)]
    ls = [jnp.zeros((M, L), jnp.int32) for _ in range(R)]
    for s in range(S):
        x = d3[:, s, :]
        xi = jnp.full((M, L), s, jnp.int32)
        for r in range(R):
            c = x < lv[r]
            nv = jnp.minimum(x, lv[r])
            xv = jnp.maximum(x, lv[r])
            ns = jnp.where(c, xi, ls[r])
            xs = jnp.where(c, ls[r], xi)
            lv[r], x = nv, xv
            ls[r], xi = ns, xs

    base = pl.program_id(0) * N
    lane = jax.lax.broadcasted_iota(jnp.int32, (M, L), 1)
    outs = []
    for _ in range(K):
        lstar = jnp.argmin(lv[0], axis=-1).astype(jnp.int32)
        oh = lane == lstar[:, None]
        sstar = jnp.take_along_axis(ls[0], lstar[:, None], axis=-1)
        outs.append((sstar[:, 0] * L + lstar + base)[None, :])
        for r in range(R - 1):
            lv[r] = jnp.where(oh, lv[r + 1], lv[r])
            ls[r] = jnp.where(oh, ls[r + 1], ls[r])
        lv[R - 1] = jnp.where(oh, INF, lv[R - 1])
    idx_ref[0] = jnp.concatenate(outs, axis=0)


def _mlp_kernel(g_ref, q_ref, wk_ref, s1_ref, h1_ref, wa_ref, wl_ref,
                bl_ref, c2_ref, out_ref):
    q = q_ref[0]
    qt = q.T
    qpad = jnp.concatenate(
        [qt, jnp.zeros((M2, CIN - 3), jnp.float32)], axis=1)
    np3 = g_ref[0] - qpad[None, :, :]
    np2 = np3.reshape(K * M2, CIN)
    kern = jax.lax.dot_general(np2, wk_ref[...], (((1,), (1,)), ((), ())),
                               precision=HIGH)
    kern = _leaky(kern * s1_ref[...] + h1_ref[...])
    wgt = jnp.sum(np2 * wa_ref[...], axis=1, keepdims=True)
    prod = kern * wgt
    acc = prod[:M2]
    for k in range(1, K):
        acc = acc + prod[k * M2:(k + 1) * M2]
    a = _leaky(acc * c2_ref[0, 0] + c2_ref[1, 0])
    out = jax.lax.dot_general(a, wl_ref[...], (((1,), (1,)), ((), ())),
                              precision=HIGH) + bl_ref[...]
    out_ref[0] = _leaky(out).T


def _gather_sc(table, indices):
    num_idx = B * K * N
    win = 128

    @pl.kernel(out_type=jax.ShapeDtypeStruct((num_idx, CIN), jnp.float32),
               mesh=plsc.VectorSubcoreMesh(core_axis_name="core",
                                           subcore_axis_name="subcore"))
    def _k(x_hbm, i_hbm, o_hbm):
        def body(i_vmem, o_vmem):
            pltpu.sync_copy(x_hbm.at[i_vmem.at[0]], o_vmem)

        pltpu.emit_pipeline(
            body,
            grid=(num_idx // win,),
            in_specs=[pl.BlockSpec((1, win), index_map=lambda i: (0, i))],
            out_specs=[pl.BlockSpec((win, CIN), index_map=lambda i: (i, 0))],
            core_axis_name=("core", "subcore"),
            dimension_semantics=(pltpu.PARALLEL,),
        )(i_hbm, o_hbm)

    return _k(table, indices)


def kernel(xyz, points, W_kernel, bn1_gamma, bn1_beta, bn1_mean, bn1_var,
           W_agg, bn2_gamma, bn2_beta, bn2_mean, bn2_var, W_lin, b_lin):
    idx = pl.pallas_call(
        _knn_kernel,
        grid=(B, N // M),
        in_specs=[
            pl.BlockSpec((1, 3, M), lambda b, i: (b, 0, i)),
            pl.BlockSpec((1, 3, N), lambda b, i: (b, 0, 0)),
        ],
        out_specs=pl.BlockSpec((1, K, M), lambda b, i: (b, 0, i)),
        out_shape=jax.ShapeDtypeStruct((B, K, N), jnp.int32),
    )(xyz, xyz)

    u = jnp.concatenate([xyz, points], axis=1)
    table = jnp.transpose(u, (0, 2, 1)).reshape(B * N, CIN)
    g = jnp.take(table, idx.reshape(B * K * N), axis=0)
    g = g.reshape(B, K, N, CIN)

    scale1 = (bn1_gamma / jnp.sqrt(bn1_var + EPS))[None, :]
    shift1 = bn1_beta[None, :] - bn1_mean[None, :] * scale1
    s2 = bn2_gamma[0] / jnp.sqrt(bn2_var[0] + EPS)
    c2 = jnp.stack([s2, bn2_beta[0] - bn2_mean[0] * s2]).reshape(2, 1)
    out = pl.pallas_call(
        _mlp_kernel,
        grid=(B, N // M2),
        in_specs=[
            pl.BlockSpec((1, K, M2, CIN), lambda b, i: (b, 0, i, 0)),
            pl.BlockSpec((1, 3, M2), lambda b, i: (b, 0, i)),
            pl.BlockSpec((O, CIN), lambda b, i: (0, 0)),
            pl.BlockSpec((1, O), lambda b, i: (0, 0)),
            pl.BlockSpec((1, O), lambda b, i: (0, 0)),
            pl.BlockSpec((1, CIN), lambda b, i: (0, 0)),
            pl.BlockSpec((O, O), lambda b, i: (0, 0)),
            pl.BlockSpec((1, O), lambda b, i: (0, 0)),
            pl.BlockSpec((2, 1), lambda b, i: (0, 0)),
        ],
        out_specs=pl.BlockSpec((1, O, M2), lambda b, i: (b, 0, i)),
        out_shape=jax.ShapeDtypeStruct((B, O, N), jnp.float32),
    )(g, xyz, W_kernel, scale1, shift1, W_agg, W_lin, b_lin[None, :], c2)
    return out


# speedup vs baseline: 9.7879x; 1.6668x over previous
import dataclasses

import jax
import jax.numpy as jnp
from jax.experimental import pallas as pl
from jax.experimental.pallas import tpu as pltpu
from jax.experimental.pallas import tpu_sc as plsc

EPS = 1e-5
LEAKY = 0.1
B, N, DF, K, CIN, O = 4, 4096, 13, 32, 16, 32
S, L = 32, 128
M = 128
M2 = 128
R = 4
INF = 3.0e38
HIGH = jax.lax.Precision.HIGHEST


def _leaky(x):
    return jnp.where(x >= 0, x, LEAKY * x)


def _knn_kernel(q_ref, xt_ref, idx_ref):
    q = q_ref[0]
    xt = xt_ref[0]
    qb = q.astype(jnp.bfloat16).astype(jnp.float32)
    xb = xt.astype(jnp.bfloat16).astype(jnp.float32)
    dot = qb[0][:, None] * xb[0][None, :]
    dot = dot + qb[1][:, None] * xb[1][None, :]
    dot = dot + qb[2][:, None] * xb[2][None, :]
    sqm = jnp.sum(q * q, axis=0)[:, None]
    sqj = jnp.sum(xt * xt, axis=0)[None, :]
    d = (-2.0 * dot + sqm) + sqj
    d3 = d.reshape(M, S, L)

    lv = [jnp.full((M, L), INF, jnp.float32) for _ in range(R)]
    ls = [jnp.zeros((M, L), jnp.int32) for _ in range(R)]
    for s in range(S):
        x = d3[:, s, :]
        xi = jnp.full((M, L), s, jnp.int32)
        for r in range(R):
            c = x < lv[r]
            nv = jnp.minimum(x, lv[r])
            xv = jnp.maximum(x, lv[r])
            ns = jnp.where(c, xi, ls[r])
            xs = jnp.where(c, ls[r], xi)
            lv[r], x = nv, xv
            ls[r], xi = ns, xs

    lane = jax.lax.broadcasted_iota(jnp.int32, (M, L), 1)
    outs = []
    for _ in range(K):
        lstar = jnp.argmin(lv[0], axis=-1).astype(jnp.int32)
        oh = lane == lstar[:, None]
        sstar = jnp.take_along_axis(ls[0], lstar[:, None], axis=-1)
        outs.append((sstar[:, 0] * L + lstar)[None, :])
        for r in range(R - 1):
            lv[r] = jnp.where(oh, lv[r + 1], lv[r])
            ls[r] = jnp.where(oh, ls[r + 1], ls[r])
        lv[R - 1] = jnp.where(oh, INF, lv[R - 1])
    idx_ref[0] = jnp.concatenate(outs, axis=0)


def _mlp_kernel(g_ref, q_ref, wk_ref, s1_ref, h1_ref, wa_ref, wl_ref,
                bl_ref, c2_ref, out_ref):
    q = q_ref[0]
    qpad = jnp.concatenate(
        [q, jnp.zeros((CIN - 3, M2), jnp.float32)], axis=0)
    wk = wk_ref[...]
    wa = wa_ref[...]
    s1 = s1_ref[...]
    h1 = h1_ref[...]
    acc = jnp.zeros((O, M2), jnp.float32)
    for k in range(K):
        np_k = g_ref[0, k] - qpad
        kern = jax.lax.dot_general(wk, np_k, (((1,), (0,)), ((), ())),
                                   precision=HIGH)
        kern = _leaky(kern * s1 + h1)
        wgt = jnp.sum(np_k * wa, axis=0, keepdims=True)
        acc = acc + kern * wgt
    a = _leaky(acc * c2_ref[0, 0] + c2_ref[1, 0])
    out = jax.lax.dot_general(wl_ref[...], a, (((1,), (0,)), ((), ())),
                              precision=HIGH) + bl_ref[...]
    out_ref[0] = _leaky(out)


NW = 1024
SC_CORES, SC_SUBS = 2, 16


def _gather_sc(u, idx):
    mesh = plsc.VectorSubcoreMesh(core_axis_name="core",
                                  subcore_axis_name="subcore")
    kper = K // 8
    cp = pltpu.CompilerParams()
    if "needs_layout_passes" in pltpu.CompilerParams.__dataclass_fields__:
        cp = dataclasses.replace(cp, needs_layout_passes=False)

    @pl.kernel(out_type=jax.ShapeDtypeStruct((B * K * CIN, N), jnp.float32),
               mesh=mesh, compiler_params=cp,
               scratch_types=[pltpu.VMEM((CIN, N), jnp.float32),
                              pltpu.VMEM((1, NW), jnp.int32),
                              pltpu.VMEM((CIN, NW), jnp.float32),
                              pltpu.SemaphoreType.DMA,
                              pltpu.SemaphoreType.DMA,
                              pltpu.SemaphoreType.DMA])
    def _k(u_hbm, i_hbm, o_hbm, tbl, iwin, owin, sem1, sem2, sem3):
        core = jax.lax.axis_index("core")
        sub = jax.lax.axis_index("subcore")
        uid = core * SC_SUBS + sub
        b = uid // 8
        kbase = (uid % 8) * kper
        pltpu.async_copy(u_hbm.at[pl.ds(b * CIN, CIN)], tbl, sem1).wait()

        @pl.loop(0, kper)
        def _kk(kk):
            bk = b * K + kbase + kk

            @pl.loop(0, N // NW)
            def _w(w):
                pltpu.async_copy(i_hbm.at[pl.ds(bk, 1), pl.ds(w * NW, NW)],
                                 iwin, sem2).wait()

                @pl.loop(0, NW // 16)
                def _t(t):
                    jvec = iwin[0, pl.ds(t * 16, 16)]
                    for c in range(CIN):
                        cvec = jnp.full((16,), c, jnp.int32)
                        owin[c, pl.ds(t * 16, 16)] = plsc.load_gather(
                            tbl, [cvec, jvec])

                pltpu.async_copy(owin,
                                 o_hbm.at[pl.ds(bk * CIN, CIN),
                                          pl.ds(w * NW, NW)],
                                 sem3).wait()

    return _k(u.reshape(B * CIN, N), idx.reshape(B * K, N)).reshape(
        B, K, CIN, N)


def kernel(xyz, points, W_kernel, bn1_gamma, bn1_beta, bn1_mean, bn1_var,
           W_agg, bn2_gamma, bn2_beta, bn2_mean, bn2_var, W_lin, b_lin):
    idx = pl.pallas_call(
        _knn_kernel,
        grid=(B, N // M),
        in_specs=[
            pl.BlockSpec((1, 3, M), lambda b, i: (b, 0, i)),
            pl.BlockSpec((1, 3, N), lambda b, i: (b, 0, 0)),
        ],
        out_specs=pl.BlockSpec((1, K, M), lambda b, i: (b, 0, i)),
        out_shape=jax.ShapeDtypeStruct((B, K, N), jnp.int32),
    )(xyz, xyz)

    u = jnp.concatenate([xyz, points], axis=1)
    g = _gather_sc(u, idx)

    scale1 = (bn1_gamma / jnp.sqrt(bn1_var + EPS))[:, None]
    shift1 = bn1_beta[:, None] - bn1_mean[:, None] * scale1
    s2 = bn2_gamma[0] / jnp.sqrt(bn2_var[0] + EPS)
    c2 = jnp.stack([s2, bn2_beta[0] - bn2_mean[0] * s2]).reshape(2, 1)
    out = pl.pallas_call(
        _mlp_kernel,
        grid=(B, N // M2),
        in_specs=[
            pl.BlockSpec((1, K, CIN, M2), lambda b, i: (b, 0, 0, i)),
            pl.BlockSpec((1, 3, M2), lambda b, i: (b, 0, i)),
            pl.BlockSpec((O, CIN), lambda b, i: (0, 0)),
            pl.BlockSpec((O, 1), lambda b, i: (0, 0)),
            pl.BlockSpec((O, 1), lambda b, i: (0, 0)),
            pl.BlockSpec((CIN, 1), lambda b, i: (0, 0)),
            pl.BlockSpec((O, O), lambda b, i: (0, 0)),
            pl.BlockSpec((O, 1), lambda b, i: (0, 0)),
            pl.BlockSpec((2, 1), lambda b, i: (0, 0)),
        ],
        out_specs=pl.BlockSpec((1, O, M2), lambda b, i: (b, 0, i)),
        out_shape=jax.ShapeDtypeStruct((B, O, N), jnp.float32),
    )(g, xyz, W_kernel, scale1, shift1, W_agg.reshape(CIN, 1),
      W_lin, b_lin[:, None], c2)
    return out


# speedup vs baseline: 19.1321x; 1.9547x over previous
import dataclasses

import jax
import jax.numpy as jnp
from jax.experimental import pallas as pl
from jax.experimental.pallas import tpu as pltpu
from jax.experimental.pallas import tpu_sc as plsc

EPS = 1e-5
LEAKY = 0.1
B, N, DF, K, CIN, O = 4, 4096, 13, 32, 16, 32
S, L = 32, 128
M = 128
MH = 64
M2 = 128
R = 4
INF = 3.0e38
HIGH = jax.lax.Precision.HIGHEST


def _leaky(x):
    return jnp.where(x >= 0, x, LEAKY * x)


def _knn_kernel(q_ref, xe4_ref, idx_ref):
    q_all = q_ref[0]
    sq_all = jnp.sum(q_all * q_all, axis=0)[None, :]
    qb_all = q_all.astype(jnp.bfloat16).astype(jnp.float32)
    lane = jax.lax.broadcasted_iota(jnp.int32, (MH, L), 1)
    halves = []
    for h in range(M // MH):
        qb = qb_all[:, h * MH:(h + 1) * MH]
        qc = [qb[c][:, None] for c in range(3)]
        sqm = sq_all[:, h * MH:(h + 1) * MH].T

        lv = [jnp.full((MH, L), INF, jnp.float32) for _ in range(R)]
        ls = [jnp.zeros((MH, L), jnp.int32) for _ in range(R)]
        for s in range(S):
            xe = xe4_ref[0, s]
            xs_ = xe.astype(jnp.bfloat16).astype(jnp.float32)
            sqj = (xe[0:1] * xe[0:1] + xe[1:2] * xe[1:2]
                   + xe[2:3] * xe[2:3])
            dot = qc[0] * xs_[0:1]
            dot = dot + qc[1] * xs_[1:2]
            dot = dot + qc[2] * xs_[2:3]
            x = (-2.0 * dot + sqm) + sqj
            xi = jnp.full((MH, L), s, jnp.int32)
            for r in range(R):
                c = x < lv[r]
                nv = jnp.minimum(x, lv[r])
                xv = jnp.maximum(x, lv[r])
                ns = jnp.where(c, xi, ls[r])
                xs2 = jnp.where(c, ls[r], xi)
                lv[r], x = nv, xv
                ls[r], xi = ns, xs2

        outs = []
        for _ in range(K):
            lstar = jnp.argmin(lv[0], axis=-1).astype(jnp.int32)
            oh = lane == lstar[:, None]
            sstar = jnp.take_along_axis(ls[0], lstar[:, None],
                                        axis=-1)
            outs.append((sstar[:, 0] * L + lstar)[None, :])
            for r in range(R - 1):
                lv[r] = jnp.where(oh, lv[r + 1], lv[r])
                ls[r] = jnp.where(oh, ls[r + 1], ls[r])
            lv[R - 1] = jnp.where(oh, INF, lv[R - 1])
        halves.append(jnp.concatenate(outs, axis=0))
    idx_ref[0] = jnp.concatenate(halves, axis=1)


def _mlp_kernel(g_ref, q_ref, wk_ref, s1_ref, h1_ref, wa_ref, wl_ref,
                bl_ref, c2_ref, out_ref):
    q = q_ref[0]
    qpad = jnp.concatenate(
        [q, jnp.zeros((CIN - 3, M2), jnp.float32)], axis=0)
    wk = wk_ref[...]
    wa = wa_ref[...]
    s1 = s1_ref[...]
    h1 = h1_ref[...]
    acc = jnp.zeros((O, M2), jnp.float32)
    for k in range(K):
        np_k = g_ref[0, k] - qpad
        kern = jax.lax.dot_general(wk, np_k, (((1,), (0,)), ((), ())),
                                   precision=HIGH)
        kern = _leaky(kern * s1 + h1)
        wgt = jnp.sum(np_k * wa, axis=0, keepdims=True)
        acc = acc + kern * wgt
    a = _leaky(acc * c2_ref[0, 0] + c2_ref[1, 0])
    out = jax.lax.dot_general(wl_ref[...], a, (((1,), (0,)), ((), ())),
                              precision=HIGH) + bl_ref[...]
    out_ref[0] = _leaky(out)


NW = 1024
SC_CORES, SC_SUBS = 2, 16


def _gather_sc(u, idx):
    mesh = plsc.VectorSubcoreMesh(core_axis_name="core",
                                  subcore_axis_name="subcore")
    kper = K // 8
    cp = pltpu.CompilerParams()
    if "needs_layout_passes" in pltpu.CompilerParams.__dataclass_fields__:
        cp = dataclasses.replace(cp, needs_layout_passes=False)

    @pl.kernel(out_type=jax.ShapeDtypeStruct((B * K * CIN, N), jnp.float32),
               mesh=mesh, compiler_params=cp,
               scratch_types=[pltpu.VMEM((CIN, N), jnp.float32),
                              pltpu.VMEM((1, NW), jnp.int32),
                              pltpu.VMEM((CIN, NW), jnp.float32),
                              pltpu.SemaphoreType.DMA,
                              pltpu.SemaphoreType.DMA,
                              pltpu.SemaphoreType.DMA])
    def _k(u_hbm, i_hbm, o_hbm, tbl, iwin, owin, sem1, sem2, sem3):
        core = jax.lax.axis_index("core")
        sub = jax.lax.axis_index("subcore")
        uid = core * SC_SUBS + sub
        b = uid // 8
        kbase = (uid % 8) * kper
        pltpu.async_copy(u_hbm.at[pl.ds(b * CIN, CIN)], tbl, sem1).wait()

        @pl.loop(0, kper)
        def _kk(kk):
            bk = b * K + kbase + kk

            @pl.loop(0, N // NW)
            def _w(w):
                pltpu.async_copy(i_hbm.at[pl.ds(bk, 1), pl.ds(w * NW, NW)],
                                 iwin, sem2).wait()

                @pl.loop(0, NW // 16)
                def _t(t):
                    jvec = iwin[0, pl.ds(t * 16, 16)]
                    for c in range(CIN):
                        cvec = jnp.full((16,), c, jnp.int32)
                        owin[c, pl.ds(t * 16, 16)] = plsc.load_gather(
                            tbl, [cvec, jvec])

                pltpu.async_copy(owin,
                                 o_hbm.at[pl.ds(bk * CIN, CIN),
                                          pl.ds(w * NW, NW)],
                                 sem3).wait()

    return _k(u.reshape(B * CIN, N), idx.reshape(B * K, N)).reshape(
        B, K, CIN, N)


def kernel(xyz, points, W_kernel, bn1_gamma, bn1_beta, bn1_mean, bn1_var,
           W_agg, bn2_gamma, bn2_beta, bn2_mean, bn2_var, W_lin, b_lin):
    xe4 = jnp.transpose(xyz.reshape(B, 3, S, L), (0, 2, 1, 3))
    idx = pl.pallas_call(
        _knn_kernel,
        grid=(B, N // M),
        in_specs=[
            pl.BlockSpec((1, 3, M), lambda b, i: (b, 0, i)),
            pl.BlockSpec((1, S, 3, L), lambda b, i: (b, 0, 0, 0)),
        ],
        out_specs=pl.BlockSpec((1, K, M), lambda b, i: (b, 0, i)),
        out_shape=jax.ShapeDtypeStruct((B, K, N), jnp.int32),
    )(xyz, xe4)

    u = jnp.concatenate([xyz, points], axis=1)
    g = _gather_sc(u, idx)

    scale1 = (bn1_gamma / jnp.sqrt(bn1_var + EPS))[:, None]
    shift1 = bn1_beta[:, None] - bn1_mean[:, None] * scale1
    s2 = bn2_gamma[0] / jnp.sqrt(bn2_var[0] + EPS)
    c2 = jnp.stack([s2, bn2_beta[0] - bn2_mean[0] * s2]).reshape(2, 1)
    out = pl.pallas_call(
        _mlp_kernel,
        grid=(B, N // M2),
        in_specs=[
            pl.BlockSpec((1, K, CIN, M2), lambda b, i: (b, 0, 0, i)),
            pl.BlockSpec((1, 3, M2), lambda b, i: (b, 0, i)),
            pl.BlockSpec((O, CIN), lambda b, i: (0, 0)),
            pl.BlockSpec((O, 1), lambda b, i: (0, 0)),
            pl.BlockSpec((O, 1), lambda b, i: (0, 0)),
            pl.BlockSpec((CIN, 1), lambda b, i: (0, 0)),
            pl.BlockSpec((O, O), lambda b, i: (0, 0)),
            pl.BlockSpec((O, 1), lambda b, i: (0, 0)),
            pl.BlockSpec((2, 1), lambda b, i: (0, 0)),
        ],
        out_specs=pl.BlockSpec((1, O, M2), lambda b, i: (b, 0, i)),
        out_shape=jax.ShapeDtypeStruct((B, O, N), jnp.float32),
    )(g, xyz, W_kernel, scale1, shift1, W_agg.reshape(CIN, 1),
      W_lin, b_lin[:, None], c2)
    return out


# speedup vs baseline: 31.9103x; 1.6679x over previous
import dataclasses

import jax
import jax.numpy as jnp
from jax.experimental import pallas as pl
from jax.experimental.pallas import tpu as pltpu
from jax.experimental.pallas import tpu_sc as plsc

EPS = 1e-5
LEAKY = 0.1
B, N, DF, K, CIN, O = 4, 4096, 13, 32, 16, 32
S, L = 32, 128
M = 128
MH = 64
M2 = 128
R = 4
INF = 3.0e38
HIGH = jax.lax.Precision.HIGHEST


def _leaky(x):
    return jnp.where(x >= 0, x, LEAKY * x)


def _knn_kernel(q_ref, xe4_ref, idx_ref):
    q_all = q_ref[0]
    sq_all = jnp.sum(q_all * q_all, axis=0)[None, :]
    qb_all = q_all.astype(jnp.bfloat16).astype(jnp.float32)
    lane = jax.lax.broadcasted_iota(jnp.int32, (MH, L), 1)
    BIGI = jnp.int32(2 ** 30)
    NHALF = M // MH
    lists = []
    for h in range(NHALF):
        qb = qb_all[:, h * MH:(h + 1) * MH]
        qc = [qb[c][:, None] for c in range(3)]
        sqm = sq_all[:, h * MH:(h + 1) * MH].T

        lv = [jnp.full((MH, L), INF, jnp.float32) for _ in range(R)]
        lj = [jnp.zeros((MH, L), jnp.int32) for _ in range(R)]
        for s in range(S):
            xe = xe4_ref[0, s]
            xs_ = xe.astype(jnp.bfloat16).astype(jnp.float32)
            sqj = (xe[0:1] * xe[0:1] + xe[1:2] * xe[1:2]
                   + xe[2:3] * xe[2:3])
            dot = qc[0] * xs_[0:1]
            dot = dot + qc[1] * xs_[1:2]
            dot = dot + qc[2] * xs_[2:3]
            x = (-2.0 * dot + sqm) + sqj
            xi = lane + s * L
            for r in range(R):
                c = x < lv[r]
                nv = jnp.minimum(x, lv[r])
                xv = jnp.maximum(x, lv[r])
                ns = jnp.where(c, xi, lj[r])
                xs2 = jnp.where(c, lj[r], xi)
                lv[r], x = nv, xv
                lj[r], xi = ns, xs2
        lists.append((lv, lj))

    outs = [[] for _ in range(NHALF)]
    for _ in range(K):
        for h in range(NHALF):
            lv, lj = lists[h]
            lstar = jnp.argmin(lv[0], axis=-1).astype(jnp.int32)
            oh = lane == lstar[:, None]
            jstar = jnp.min(jnp.where(oh, lj[0], BIGI), axis=-1)
            outs[h].append(jstar[None, :])
            for r in range(R - 1):
                lv[r] = jnp.where(oh, lv[r + 1], lv[r])
                lj[r] = jnp.where(oh, lj[r + 1], lj[r])
            lv[R - 1] = jnp.where(oh, INF, lv[R - 1])
    idx_ref[0] = jnp.concatenate(
        [jnp.concatenate(outs[h], axis=0) for h in range(NHALF)], axis=1)


def _mlp_kernel(g_ref, q_ref, wk_ref, s1_ref, h1_ref, wa_ref, wl_ref,
                bl_ref, c2_ref, out_ref):
    q = q_ref[0]
    qpad = jnp.concatenate(
        [q, jnp.zeros((CIN - 3, M2), jnp.float32)], axis=0)
    wk = wk_ref[...]
    wa = wa_ref[...]
    s1 = s1_ref[...]
    h1 = h1_ref[...]
    acc = jnp.zeros((O, M2), jnp.float32)
    for k in range(K):
        np_k = g_ref[0, k] - qpad
        kern = jax.lax.dot_general(wk, np_k, (((1,), (0,)), ((), ())),
                                   precision=HIGH)
        kern = _leaky(kern * s1 + h1)
        wgt = jnp.sum(np_k * wa, axis=0, keepdims=True)
        acc = acc + kern * wgt
    a = _leaky(acc * c2_ref[0, 0] + c2_ref[1, 0])
    out = jax.lax.dot_general(wl_ref[...], a, (((1,), (0,)), ((), ())),
                              precision=HIGH) + bl_ref[...]
    out_ref[0] = _leaky(out)


NW = 1024
SC_CORES, SC_SUBS = 2, 16


def _gather_sc(u, idx):
    mesh = plsc.VectorSubcoreMesh(core_axis_name="core",
                                  subcore_axis_name="subcore")
    kper = K // 8
    cp = pltpu.CompilerParams()
    if "needs_layout_passes" in pltpu.CompilerParams.__dataclass_fields__:
        cp = dataclasses.replace(cp, needs_layout_passes=False)

    @pl.kernel(out_type=jax.ShapeDtypeStruct((B * K * CIN, N), jnp.float32),
               mesh=mesh, compiler_params=cp,
               scratch_types=[pltpu.VMEM((CIN, N), jnp.float32),
                              pltpu.VMEM((1, NW), jnp.int32),
                              pltpu.VMEM((CIN, NW), jnp.float32),
                              pltpu.SemaphoreType.DMA,
                              pltpu.SemaphoreType.DMA,
                              pltpu.SemaphoreType.DMA])
    def _k(u_hbm, i_hbm, o_hbm, tbl, iwin, owin, sem1, sem2, sem3):
        core = jax.lax.axis_index("core")
        sub = jax.lax.axis_index("subcore")
        uid = core * SC_SUBS + sub
        b = uid // 8
        kbase = (uid % 8) * kper
        pltpu.async_copy(u_hbm.at[pl.ds(b * CIN, CIN)], tbl, sem1).wait()

        @pl.loop(0, kper)
        def _kk(kk):
            bk = b * K + kbase + kk

            @pl.loop(0, N // NW)
            def _w(w):
                pltpu.async_copy(i_hbm.at[pl.ds(bk, 1), pl.ds(w * NW, NW)],
                                 iwin, sem2).wait()

                @pl.loop(0, NW // 16)
                def _t(t):
                    jvec = iwin[0, pl.ds(t * 16, 16)]
                    for c in range(CIN):
                        cvec = jnp.full((16,), c, jnp.int32)
                        owin[c, pl.ds(t * 16, 16)] = plsc.load_gather(
                            tbl, [cvec, jvec])

                pltpu.async_copy(owin,
                                 o_hbm.at[pl.ds(bk * CIN, CIN),
                                          pl.ds(w * NW, NW)],
                                 sem3).wait()

    return _k(u.reshape(B * CIN, N), idx.reshape(B * K, N)).reshape(
        B, K, CIN, N)


def kernel(xyz, points, W_kernel, bn1_gamma, bn1_beta, bn1_mean, bn1_var,
           W_agg, bn2_gamma, bn2_beta, bn2_mean, bn2_var, W_lin, b_lin):
    xe4 = jnp.transpose(xyz.reshape(B, 3, S, L), (0, 2, 1, 3))
    idx = pl.pallas_call(
        _knn_kernel,
        grid=(B, N // M),
        in_specs=[
            pl.BlockSpec((1, 3, M), lambda b, i: (b, 0, i)),
            pl.BlockSpec((1, S, 3, L), lambda b, i: (b, 0, 0, 0)),
        ],
        out_specs=pl.BlockSpec((1, K, M), lambda b, i: (b, 0, i)),
        out_shape=jax.ShapeDtypeStruct((B, K, N), jnp.int32),
    )(xyz, xe4)

    u = jnp.concatenate([xyz, points], axis=1)
    g = _gather_sc(u, idx)

    scale1 = (bn1_gamma / jnp.sqrt(bn1_var + EPS))[:, None]
    shift1 = bn1_beta[:, None] - bn1_mean[:, None] * scale1
    s2 = bn2_gamma[0] / jnp.sqrt(bn2_var[0] + EPS)
    c2 = jnp.stack([s2, bn2_beta[0] - bn2_mean[0] * s2]).reshape(2, 1)
    out = pl.pallas_call(
        _mlp_kernel,
        grid=(B, N // M2),
        in_specs=[
            pl.BlockSpec((1, K, CIN, M2), lambda b, i: (b, 0, 0, i)),
            pl.BlockSpec((1, 3, M2), lambda b, i: (b, 0, i)),
            pl.BlockSpec((O, CIN), lambda b, i: (0, 0)),
            pl.BlockSpec((O, 1), lambda b, i: (0, 0)),
            pl.BlockSpec((O, 1), lambda b, i: (0, 0)),
            pl.BlockSpec((CIN, 1), lambda b, i: (0, 0)),
            pl.BlockSpec((O, O), lambda b, i: (0, 0)),
            pl.BlockSpec((O, 1), lambda b, i: (0, 0)),
            pl.BlockSpec((2, 1), lambda b, i: (0, 0)),
        ],
        out_specs=pl.BlockSpec((1, O, M2), lambda b, i: (b, 0, i)),
        out_shape=jax.ShapeDtypeStruct((B, O, N), jnp.float32),
    )(g, xyz, W_kernel, scale1, shift1, W_agg.reshape(CIN, 1),
      W_lin, b_lin[:, None], c2)
    return out


# speedup vs baseline: 32.0786x; 1.0053x over previous
import dataclasses

import jax
import jax.numpy as jnp
from jax.experimental import pallas as pl
from jax.experimental.pallas import tpu as pltpu
from jax.experimental.pallas import tpu_sc as plsc

EPS = 1e-5
LEAKY = 0.1
B, N, DF, K, CIN, O = 4, 4096, 13, 32, 16, 32
S, L = 32, 128
M = 128
MH = 64
M2 = 128
R = 4
INF = 3.0e38
HIGH = jax.lax.Precision.HIGHEST


def _leaky(x):
    return jnp.where(x >= 0, x, LEAKY * x)


def _knn_kernel(q_ref, xe4_ref, idx_ref):
    q_all = q_ref[0]
    sq_all = jnp.sum(q_all * q_all, axis=0)[None, :]
    qb_all = q_all.astype(jnp.bfloat16).astype(jnp.float32)
    lane = jax.lax.broadcasted_iota(jnp.int32, (MH, L), 1)
    BIGI = jnp.int32(2 ** 30)
    NHALF = M // MH
    lists = []
    for h in range(NHALF):
        qb = qb_all[:, h * MH:(h + 1) * MH]
        qc = [qb[c][:, None] for c in range(3)]
        sqm = sq_all[:, h * MH:(h + 1) * MH].T

        lv = [jnp.full((MH, L), INF, jnp.float32) for _ in range(R)]
        lj = [jnp.zeros((MH, L), jnp.int32) for _ in range(R)]
        for s in range(S):
            xe = xe4_ref[0, s]
            xs_ = xe.astype(jnp.bfloat16).astype(jnp.float32)
            sqj = (xe[0:1] * xe[0:1] + xe[1:2] * xe[1:2]
                   + xe[2:3] * xe[2:3])
            dot = qc[0] * xs_[0:1]
            dot = dot + qc[1] * xs_[1:2]
            dot = dot + qc[2] * xs_[2:3]
            x = (-2.0 * dot + sqm) + sqj
            xi = lane + s * L
            for r in range(R):
                c = x < lv[r]
                nv = jnp.minimum(x, lv[r])
                xv = jnp.maximum(x, lv[r])
                ns = jnp.where(c, xi, lj[r])
                xs2 = jnp.where(c, lj[r], xi)
                lv[r], x = nv, xv
                lj[r], xi = ns, xs2
        lists.append((lv, lj))

    outs = [[] for _ in range(NHALF)]
    for _ in range(K):
        for h in range(NHALF):
            lv, lj = lists[h]
            lstar = jnp.argmin(lv[0], axis=-1).astype(jnp.int32)
            oh = lane == lstar[:, None]
            jstar = jnp.min(jnp.where(oh, lj[0], BIGI), axis=-1)
            outs[h].append(jstar[None, :])
            for r in range(R - 1):
                lv[r] = jnp.where(oh, lv[r + 1], lv[r])
                lj[r] = jnp.where(oh, lj[r + 1], lj[r])
            lv[R - 1] = jnp.where(oh, INF, lv[R - 1])
    idx_ref[0] = jnp.concatenate(
        [jnp.concatenate(outs[h], axis=0) for h in range(NHALF)], axis=1)


def _mlp_kernel(g_ref, q_ref, wk_ref, s1_ref, h1_ref, wa_ref, wl_ref,
                bl_ref, c2_ref, out_ref):
    q = q_ref[0]
    qpad = jnp.concatenate(
        [q, jnp.zeros((CIN - 3, M2), jnp.float32)], axis=0)
    wk = wk_ref[...]
    wa = wa_ref[...]
    s1 = s1_ref[...]
    h1 = h1_ref[...]
    acc = jnp.zeros((O, M2), jnp.float32)
    for k in range(K):
        np_k = g_ref[0, k] - qpad
        kern = jax.lax.dot_general(wk, np_k, (((1,), (0,)), ((), ())),
                                   precision=HIGH)
        kern = _leaky(kern * s1 + h1)
        wgt = jnp.sum(np_k * wa, axis=0, keepdims=True)
        acc = acc + kern * wgt
    a = _leaky(acc * c2_ref[0, 0] + c2_ref[1, 0])
    out = jax.lax.dot_general(wl_ref[...], a, (((1,), (0,)), ((), ())),
                              precision=HIGH) + bl_ref[...]
    out_ref[0] = _leaky(out)


NW = 2048
SC_CORES, SC_SUBS = 2, 16


def _gather_sc(u, idx):
    mesh = plsc.VectorSubcoreMesh(core_axis_name="core",
                                  subcore_axis_name="subcore")
    kper = K // 8
    cp = pltpu.CompilerParams()
    if "needs_layout_passes" in pltpu.CompilerParams.__dataclass_fields__:
        cp = dataclasses.replace(cp, needs_layout_passes=False)

    @pl.kernel(out_type=jax.ShapeDtypeStruct((B * K * CIN, N), jnp.float32),
               mesh=mesh, compiler_params=cp,
               scratch_types=[pltpu.VMEM((CIN, N), jnp.float32),
                              pltpu.VMEM((1, NW), jnp.int32),
                              pltpu.VMEM((CIN, NW), jnp.float32),
                              pltpu.SemaphoreType.DMA,
                              pltpu.SemaphoreType.DMA,
                              pltpu.SemaphoreType.DMA])
    def _k(u_hbm, i_hbm, o_hbm, tbl, iwin, owin, sem1, sem2, sem3):
        core = jax.lax.axis_index("core")
        sub = jax.lax.axis_index("subcore")
        uid = core * SC_SUBS + sub
        b = uid // 8
        kbase = (uid % 8) * kper
        pltpu.async_copy(u_hbm.at[pl.ds(b * CIN, CIN)], tbl, sem1).wait()

        @pl.loop(0, kper)
        def _kk(kk):
            bk = b * K + kbase + kk

            @pl.loop(0, N // NW)
            def _w(w):
                pltpu.async_copy(i_hbm.at[pl.ds(bk, 1), pl.ds(w * NW, NW)],
                                 iwin, sem2).wait()

                @pl.loop(0, NW // 16)
                def _t(t):
                    jvec = iwin[0, pl.ds(t * 16, 16)]
                    for c in range(CIN):
                        cvec = jnp.full((16,), c, jnp.int32)
                        owin[c, pl.ds(t * 16, 16)] = plsc.load_gather(
                            tbl, [cvec, jvec])

                pltpu.async_copy(owin,
                                 o_hbm.at[pl.ds(bk * CIN, CIN),
                                          pl.ds(w * NW, NW)],
                                 sem3).wait()

    return _k(u.reshape(B * CIN, N), idx.reshape(B * K, N)).reshape(
        B, K, CIN, N)


def kernel(xyz, points, W_kernel, bn1_gamma, bn1_beta, bn1_mean, bn1_var,
           W_agg, bn2_gamma, bn2_beta, bn2_mean, bn2_var, W_lin, b_lin):
    xe4 = jnp.transpose(xyz.reshape(B, 3, S, L), (0, 2, 1, 3))
    idx = pl.pallas_call(
        _knn_kernel,
        grid=(B, N // M),
        in_specs=[
            pl.BlockSpec((1, 3, M), lambda b, i: (b, 0, i)),
            pl.BlockSpec((1, S, 3, L), lambda b, i: (b, 0, 0, 0)),
        ],
        out_specs=pl.BlockSpec((1, K, M), lambda b, i: (b, 0, i)),
        out_shape=jax.ShapeDtypeStruct((B, K, N), jnp.int32),
    )(xyz, xe4)

    u = jnp.concatenate([xyz, points], axis=1)
    g = _gather_sc(u, idx)

    scale1 = (bn1_gamma / jnp.sqrt(bn1_var + EPS))[:, None]
    shift1 = bn1_beta[:, None] - bn1_mean[:, None] * scale1
    s2 = bn2_gamma[0] / jnp.sqrt(bn2_var[0] + EPS)
    c2 = jnp.stack([s2, bn2_beta[0] - bn2_mean[0] * s2]).reshape(2, 1)
    out = pl.pallas_call(
        _mlp_kernel,
        grid=(B, N // M2),
        in_specs=[
            pl.BlockSpec((1, K, CIN, M2), lambda b, i: (b, 0, 0, i)),
            pl.BlockSpec((1, 3, M2), lambda b, i: (b, 0, i)),
            pl.BlockSpec((O, CIN), lambda b, i: (0, 0)),
            pl.BlockSpec((O, 1), lambda b, i: (0, 0)),
            pl.BlockSpec((O, 1), lambda b, i: (0, 0)),
            pl.BlockSpec((CIN, 1), lambda b, i: (0, 0)),
            pl.BlockSpec((O, O), lambda b, i: (0, 0)),
            pl.BlockSpec((O, 1), lambda b, i: (0, 0)),
            pl.BlockSpec((2, 1), lambda b, i: (0, 0)),
        ],
        out_specs=pl.BlockSpec((1, O, M2), lambda b, i: (b, 0, i)),
        out_shape=jax.ShapeDtypeStruct((B, O, N), jnp.float32),
    )(g, xyz, W_kernel, scale1, shift1, W_agg.reshape(CIN, 1),
      W_lin, b_lin[:, None], c2)
    return out


# speedup vs baseline: 32.1665x; 1.0027x over previous
import dataclasses

import jax
import jax.numpy as jnp
from jax.experimental import pallas as pl
from jax.experimental.pallas import tpu as pltpu
from jax.experimental.pallas import tpu_sc as plsc

EPS = 1e-5
LEAKY = 0.1
B, N, DF, K, CIN, O = 4, 4096, 13, 32, 16, 32
S, L = 32, 128
M = 128
MH = 64
M2 = 128
R = 4
INF = 3.0e38
HIGH = jax.lax.Precision.HIGHEST


def _leaky(x):
    return jnp.where(x >= 0, x, LEAKY * x)


def _knn_kernel(q_ref, xt_ref, idx_ref):
    q_all = q_ref[0]
    sq_all = jnp.sum(q_all * q_all, axis=0)[None, :]
    qb_all = q_all.astype(jnp.bfloat16).astype(jnp.float32)
    lane = jax.lax.broadcasted_iota(jnp.int32, (MH, L), 1)
    BIGI = jnp.int32(2 ** 30)
    NHALF = M // MH
    lists = []
    for h in range(NHALF):
        qb = qb_all[:, h * MH:(h + 1) * MH]
        qc = [qb[c][:, None] for c in range(3)]
        sqm = sq_all[:, h * MH:(h + 1) * MH].T

        lv = [jnp.full((MH, L), INF, jnp.float32) for _ in range(R)]
        lj = [jnp.zeros((MH, L), jnp.int32) for _ in range(R)]
        for s in range(S):
            xe = xt_ref[0][:, s * L:(s + 1) * L]
            xs_ = xe.astype(jnp.bfloat16).astype(jnp.float32)
            sqj = (xe[0:1] * xe[0:1] + xe[1:2] * xe[1:2]
                   + xe[2:3] * xe[2:3])
            dot = qc[0] * xs_[0:1]
            dot = dot + qc[1] * xs_[1:2]
            dot = dot + qc[2] * xs_[2:3]
            x = (-2.0 * dot + sqm) + sqj
            xi = lane + s * L
            for r in range(R):
                c = x < lv[r]
                nv = jnp.minimum(x, lv[r])
                xv = jnp.maximum(x, lv[r])
                ns = jnp.where(c, xi, lj[r])
                xs2 = jnp.where(c, lj[r], xi)
                lv[r], x = nv, xv
                lj[r], xi = ns, xs2
        lists.append((lv, lj))

    outs = [[] for _ in range(NHALF)]
    for _ in range(K):
        for h in range(NHALF):
            lv, lj = lists[h]
            lstar = jnp.argmin(lv[0], axis=-1).astype(jnp.int32)
            oh = lane == lstar[:, None]
            jstar = jnp.min(jnp.where(oh, lj[0], BIGI), axis=-1)
            outs[h].append(jstar[None, :])
            for r in range(R - 1):
                lv[r] = jnp.where(oh, lv[r + 1], lv[r])
                lj[r] = jnp.where(oh, lj[r + 1], lj[r])
            lv[R - 1] = jnp.where(oh, INF, lv[R - 1])
    idx_ref[0] = jnp.concatenate(
        [jnp.concatenate(outs[h], axis=0) for h in range(NHALF)], axis=1)


def _mlp_kernel(g_ref, q_ref, wk_ref, s1_ref, h1_ref, wa_ref, wl_ref,
                bl_ref, c2_ref, out_ref):
    q = q_ref[0]
    qpad = jnp.concatenate(
        [q, jnp.zeros((CIN - 3, M2), jnp.float32)], axis=0)
    wk = wk_ref[...]
    wa = wa_ref[...]
    s1 = s1_ref[...]
    h1 = h1_ref[...]
    acc = jnp.zeros((O, M2), jnp.float32)
    for k in range(K):
        np_k = g_ref[0, k] - qpad
        kern = jax.lax.dot_general(wk, np_k, (((1,), (0,)), ((), ())),
                                   precision=HIGH)
        kern = _leaky(kern * s1 + h1)
        wgt = jnp.sum(np_k * wa, axis=0, keepdims=True)
        acc = acc + kern * wgt
    a = _leaky(acc * c2_ref[0, 0] + c2_ref[1, 0])
    out = jax.lax.dot_general(wl_ref[...], a, (((1,), (0,)), ((), ())),
                              precision=HIGH) + bl_ref[...]
    out_ref[0] = _leaky(out)


NW = 2048
SC_CORES, SC_SUBS = 2, 16


def _gather_sc(u, idx):
    mesh = plsc.VectorSubcoreMesh(core_axis_name="core",
                                  subcore_axis_name="subcore")
    kper = K // 8
    cp = pltpu.CompilerParams()
    if "needs_layout_passes" in pltpu.CompilerParams.__dataclass_fields__:
        cp = dataclasses.replace(cp, needs_layout_passes=False)

    @pl.kernel(out_type=jax.ShapeDtypeStruct((B * K * CIN, N), jnp.float32),
               mesh=mesh, compiler_params=cp,
               scratch_types=[pltpu.VMEM((CIN, N), jnp.float32),
                              pltpu.VMEM((1, NW), jnp.int32),
                              pltpu.VMEM((CIN, NW), jnp.float32),
                              pltpu.SemaphoreType.DMA,
                              pltpu.SemaphoreType.DMA,
                              pltpu.SemaphoreType.DMA])
    def _k(u_hbm, i_hbm, o_hbm, tbl, iwin, owin, sem1, sem2, sem3):
        core = jax.lax.axis_index("core")
        sub = jax.lax.axis_index("subcore")
        uid = core * SC_SUBS + sub
        b = uid // 8
        kbase = (uid % 8) * kper
        pltpu.async_copy(u_hbm.at[pl.ds(b * CIN, CIN)], tbl, sem1).wait()

        @pl.loop(0, kper)
        def _kk(kk):
            bk = b * K + kbase + kk

            @pl.loop(0, N // NW)
            def _w(w):
                pltpu.async_copy(i_hbm.at[pl.ds(bk, 1), pl.ds(w * NW, NW)],
                                 iwin, sem2).wait()

                @pl.loop(0, NW // 16)
                def _t(t):
                    jvec = iwin[0, pl.ds(t * 16, 16)]
                    for c in range(CIN):
                        cvec = jnp.full((16,), c, jnp.int32)
                        owin[c, pl.ds(t * 16, 16)] = plsc.load_gather(
                            tbl, [cvec, jvec])

                pltpu.async_copy(owin,
                                 o_hbm.at[pl.ds(bk * CIN, CIN),
                                          pl.ds(w * NW, NW)],
                                 sem3).wait()

    return _k(u.reshape(B * CIN, N), idx.reshape(B * K, N)).reshape(
        B, K, CIN, N)


def kernel(xyz, points, W_kernel, bn1_gamma, bn1_beta, bn1_mean, bn1_var,
           W_agg, bn2_gamma, bn2_beta, bn2_mean, bn2_var, W_lin, b_lin):
    idx = pl.pallas_call(
        _knn_kernel,
        grid=(B, N // M),
        in_specs=[
            pl.BlockSpec((1, 3, M), lambda b, i: (b, 0, i)),
            pl.BlockSpec((1, 3, N), lambda b, i: (b, 0, 0)),
        ],
        out_specs=pl.BlockSpec((1, K, M), lambda b, i: (b, 0, i)),
        out_shape=jax.ShapeDtypeStruct((B, K, N), jnp.int32),
    )(xyz, xyz)

    u = jnp.concatenate([xyz, points], axis=1)
    g = _gather_sc(u, idx)

    scale1 = (bn1_gamma / jnp.sqrt(bn1_var + EPS))[:, None]
    shift1 = bn1_beta[:, None] - bn1_mean[:, None] * scale1
    s2 = bn2_gamma[0] / jnp.sqrt(bn2_var[0] + EPS)
    c2 = jnp.stack([s2, bn2_beta[0] - bn2_mean[0] * s2]).reshape(2, 1)
    out = pl.pallas_call(
        _mlp_kernel,
        grid=(B, N // M2),
        in_specs=[
            pl.BlockSpec((1, K, CIN, M2), lambda b, i: (b, 0, 0, i)),
            pl.BlockSpec((1, 3, M2), lambda b, i: (b, 0, i)),
            pl.BlockSpec((O, CIN), lambda b, i: (0, 0)),
            pl.BlockSpec((O, 1), lambda b, i: (0, 0)),
            pl.BlockSpec((O, 1), lambda b, i: (0, 0)),
            pl.BlockSpec((CIN, 1), lambda b, i: (0, 0)),
            pl.BlockSpec((O, O), lambda b, i: (0, 0)),
            pl.BlockSpec((O, 1), lambda b, i: (0, 0)),
            pl.BlockSpec((2, 1), lambda b, i: (0, 0)),
        ],
        out_specs=pl.BlockSpec((1, O, M2), lambda b, i: (b, 0, i)),
        out_shape=jax.ShapeDtypeStruct((B, O, N), jnp.float32),
    )(g, xyz, W_kernel, scale1, shift1, W_agg.reshape(CIN, 1),
      W_lin, b_lin[:, None], c2)
    return out
